# baseline (speedup 1.0000x reference)
import jax
import jax.numpy as jnp
from jax import lax
from jax.experimental import pallas as pl
from jax.experimental.pallas import tpu as pltpu
from jax.experimental.pallas import tpu_sc as plsc

_VOCAB = 100000
_EMBED = 64
_TOKENS = 200
_BATCH = 4096

_NC = 2
_NS = 16
_NW = _NC * _NS

_BT = _BATCH // 128
_TT = _TOKENS // 8


def _transpose_add(t, rows_v, obuf, pos_v):
    @plsc.parallel_loop(0, 8, unroll=2)
    def _blk(dt):
        seg = pos_v[t, pl.ds((dt // 2) * 16, 16)]
        lane0 = (dt % 2) * 8
        for dd in range(8):
            r = dt * 8 + dd
            p = seg.at[jnp.full((16,), lane0 + dd, jnp.int32)].get(
                mode="promise_in_bounds")
            col = jnp.full((16,), r, jnp.int32)
            for g in range(8):
                bidx = jnp.arange(16, dtype=jnp.int32) + (g * 16)
                vals = plsc.load_gather(rows_v, [bidx, col])
                obuf[dt, dd, pl.ds(g * 16, 16)] = vals + p


def _body(x_hbm, tab_hbm, pos_hbm, out_hbm,
          xbuf, pos_v, rows0, rows1, ob0, ob1,
          gsem0, gsem1, osem0, osem1):
    wid = lax.axis_index("s") * _NC + lax.axis_index("c")
    pltpu.sync_copy(pos_hbm, pos_v)
    pltpu.sync_copy(x_hbm.at[:, wid], xbuf)

    def _gather(t, rows, gsem):
        idx = xbuf.at[t // 8, t % 8]
        pltpu.async_copy(tab_hbm.at[idx], rows, gsem)

    def _drain_rows(rows, sem):
        pltpu.make_async_copy(tab_hbm.at[pl.ds(0, 128)], rows, sem).wait()

    def _drain_ob(obuf, sem):
        pltpu.make_async_copy(out_hbm.at[0, :, 0], obuf, sem).wait()

    def _emit_out(t, obuf, osem):
        pltpu.async_copy(obuf, out_hbm.at[t, :, wid], osem)

    _gather(0, rows0, gsem0)

    @pl.loop(0, _TOKENS, step=2)
    def _t2(t):
        _gather(t + 1, rows1, gsem1)
        _drain_rows(rows0, gsem0)

        @pl.when(t >= 2)
        def _():
            _drain_ob(ob0, osem0)

        _transpose_add(t, rows0, ob0, pos_v)
        _emit_out(t, ob0, osem0)

        @pl.when(t + 2 < _TOKENS)
        def _():
            _gather(t + 2, rows0, gsem0)

        _drain_rows(rows1, gsem1)

        @pl.when(t >= 2)
        def _():
            _drain_ob(ob1, osem1)

        _transpose_add(t + 1, rows1, ob1, pos_v)
        _emit_out(t + 1, ob1, osem1)

    _drain_ob(ob0, osem0)
    _drain_ob(ob1, osem1)


@jax.jit
def _run(x_lin, table, pos):
    mesh = plsc.VectorSubcoreMesh(core_axis_name="c", subcore_axis_name="s")
    kfn = pl.kernel(
        _body,
        out_type=jax.ShapeDtypeStruct((_TOKENS, 8, _BT, 8, 128), jnp.float32),
        mesh=mesh,
        scratch_types=[
            pltpu.VMEM((_TT, 8, 128), jnp.int32),
            pltpu.VMEM((_TOKENS, _EMBED), jnp.float32),
            pltpu.VMEM((128, _EMBED), jnp.float32),
            pltpu.VMEM((128, _EMBED), jnp.float32),
            pltpu.VMEM((8, 8, 128), jnp.float32),
            pltpu.VMEM((8, 8, 128), jnp.float32),
            pltpu.SemaphoreType.DMA,
            pltpu.SemaphoreType.DMA,
            pltpu.SemaphoreType.DMA,
            pltpu.SemaphoreType.DMA,
        ],
        compiler_params=pltpu.CompilerParams(use_tc_tiling_on_sc=False,
                                               needs_layout_passes=False),
    )
    return kfn(x_lin, table, pos)


def kernel(x, token_embedding, position_embedding):
    x_lin = x.astype(jnp.int32).reshape(_BT, 128, _TT, 8).transpose(2, 0, 3, 1)
    out_lin = _run(x_lin, token_embedding, position_embedding)
    return out_lin.transpose(2, 4, 0, 1, 3).reshape(_BATCH, _TOKENS, _EMBED)


# speedup vs baseline: 3.7405x; 3.7405x over previous
import jax
import jax.numpy as jnp
from jax import lax
from jax.experimental import pallas as pl
from jax.experimental.pallas import tpu as pltpu
from jax.experimental.pallas import tpu_sc as plsc

_VOCAB = 100000
_EMBED = 64
_TOKENS = 200
_BATCH = 4096

_NC = 2
_NS = 16
_NW = _NC * _NS

_BT = _BATCH // 128
_TT = _TOKENS // 8


def _transpose_add(t, rows_v, obuf, pos_v):
    iota = jnp.arange(16, dtype=jnp.int32)
    for q in range(4):
        pos_seg = pos_v[t, pl.ds(q * 16, 16)]

        @plsc.parallel_loop(0, 16, unroll=2)
        def _k(k):
            pat = (iota + k) & 15
            dvec = pat + (q * 16)
            pp = pos_seg.at[pat].get(mode="promise_in_bounds")
            for g in range(8):
                bvec = iota + (g * 16)
                vals = plsc.load_gather(rows_v, [bvec, dvec])
                plsc.store_scatter(obuf, [dvec, bvec], vals + pp)


def _body(x_hbm, tab_hbm, pos_hbm, out_hbm,
          xbuf, pos_v, rows0, rows1, ob0, ob1,
          gsem0, gsem1, osem0, osem1):
    wid = lax.axis_index("s") * _NC + lax.axis_index("c")
    pltpu.sync_copy(pos_hbm, pos_v)
    pltpu.sync_copy(x_hbm.at[:, wid], xbuf)

    def _gather(t, rows, gsem):
        idx = xbuf.at[t // 8, t % 8]
        pltpu.async_copy(tab_hbm.at[idx], rows, gsem)

    def _drain_rows(rows, sem):
        pltpu.make_async_copy(tab_hbm.at[pl.ds(0, 128)], rows, sem).wait()

    def _drain_ob(obuf, sem):
        for k in range(8):
            pltpu.make_async_copy(out_hbm.at[0, k, 0],
                                  obuf.at[pl.ds(k * 8, 8)], sem).wait()

    def _emit_out(t, obuf, osem):
        for k in range(8):
            pltpu.async_copy(obuf.at[pl.ds(k * 8, 8)], out_hbm.at[t, k, wid], osem)

    _gather(0, rows0, gsem0)

    @pl.loop(0, _TOKENS, step=2)
    def _t2(t):
        _gather(t + 1, rows1, gsem1)
        _drain_rows(rows0, gsem0)

        @pl.when(t >= 2)
        def _():
            _drain_ob(ob0, osem0)

        _transpose_add(t, rows0, ob0, pos_v)
        _emit_out(t, ob0, osem0)

        @pl.when(t + 2 < _TOKENS)
        def _():
            _gather(t + 2, rows0, gsem0)

        _drain_rows(rows1, gsem1)

        @pl.when(t >= 2)
        def _():
            _drain_ob(ob1, osem1)

        _transpose_add(t + 1, rows1, ob1, pos_v)
        _emit_out(t + 1, ob1, osem1)

    _drain_ob(ob0, osem0)
    _drain_ob(ob1, osem1)


@jax.jit
def _run(x_lin, table, pos):
    mesh = plsc.VectorSubcoreMesh(core_axis_name="c", subcore_axis_name="s")
    kfn = pl.kernel(
        _body,
        out_type=jax.ShapeDtypeStruct((_TOKENS, 8, _BT, 8, 128), jnp.float32),
        mesh=mesh,
        scratch_types=[
            pltpu.VMEM((_TT, 8, 128), jnp.int32),
            pltpu.VMEM((_TOKENS, _EMBED), jnp.float32),
            pltpu.VMEM((128, _EMBED), jnp.float32),
            pltpu.VMEM((128, _EMBED), jnp.float32),
            pltpu.VMEM((_EMBED, 128), jnp.float32),
            pltpu.VMEM((_EMBED, 128), jnp.float32),
            pltpu.SemaphoreType.DMA,
            pltpu.SemaphoreType.DMA,
            pltpu.SemaphoreType.DMA,
            pltpu.SemaphoreType.DMA,
        ],
        compiler_params=pltpu.CompilerParams(use_tc_tiling_on_sc=False,
                                               needs_layout_passes=False),
    )
    return kfn(x_lin, table, pos)


def kernel(x, token_embedding, position_embedding):
    x_lin = x.astype(jnp.int32).reshape(_BT, 128, _TT, 8).transpose(2, 0, 3, 1)
    out_lin = _run(x_lin, token_embedding, position_embedding)
    return out_lin.transpose(2, 4, 0, 1, 3).reshape(_BATCH, _TOKENS, _EMBED)
